# Initial kernel scaffold; baseline (speedup 1.0000x reference)
import jax
import jax.numpy as jnp
from jax.experimental import pallas as pl


def kernel(x, padding_mask, W1, b1, gamma, beta, W2, b2):
    raise NotImplementedError("write your pallas kernel here")


# speedup vs baseline: 1.4331x; 1.4331x over previous
import functools

import jax
import jax.numpy as jnp
from jax.experimental import pallas as pl
from jax.experimental.pallas import tpu as pltpu

D_MODEL = 512
DF = 128
LN_EPS = 1e-5

BB = 8
TB = 512


def _detector_kernel(x_ref, m_ref, w1_ref, b1_ref, gamma_ref, beta_ref,
                     w2_ref, b2_ref, o_ref,
                     s1e, s1l, s2, cnt, mx, pk, *, t_blocks, n_early):
    phase = pl.program_id(1)
    ti = pl.program_id(2)

    xb = x_ref[...]
    valid = 1.0 - m_ref[...]

    @pl.when(phase == 0)
    def _accumulate():
        @pl.when(ti == 0)
        def _init():
            s1e[...] = jnp.zeros_like(s1e)
            s1l[...] = jnp.zeros_like(s1l)
            s2[...] = jnp.zeros_like(s2)
            cnt[...] = jnp.zeros_like(cnt)
            mx[...] = jnp.full_like(mx, -65000.0)

        xv = xb * valid[:, :, None]
        s1_blk = jnp.sum(xv, axis=1)

        @pl.when(ti < n_early)
        def _():
            s1e[...] += s1_blk

        @pl.when(ti >= n_early)
        def _():
            s1l[...] += s1_blk

        s2[...] += jnp.sum(xv * xb, axis=1)
        cnt[...] += jnp.sum(valid, axis=1, keepdims=True)
        x_masked = jnp.where(m_ref[...][:, :, None] > 0.0, -65000.0, xb)
        mx[...] = jnp.maximum(mx[...],
                              jnp.max(x_masked, axis=(1, 2), keepdims=False)[:, None])

    @pl.when(phase == 1)
    def _peaks():
        denom = cnt[...] + 1e-8
        s1 = s1e[...] + s1l[...]
        pooled = s1 / denom
        thr = jnp.max(pooled, axis=-1, keepdims=True) * 0.7

        @pl.when(ti == 0)
        def _init():
            pk[...] = jnp.zeros_like(pk)

        high = (xb > thr[:, :, None]).astype(jnp.float32)
        pk[...] += jnp.sum(high * valid[:, :, None], axis=1)

        @pl.when(ti == t_blocks - 1)
        def _head():
            max_strength = mx[...]
            x_var = (s2[...] - 2.0 * pooled * s1
                     + pooled * pooled * cnt[...]) / denom
            variance = jnp.max(x_var, axis=-1, keepdims=True)
            peak_count = jnp.max(pk[...], axis=-1, keepdims=True)
            early = jnp.max(s1e[...], axis=-1, keepdims=True)
            late = jnp.max(s1l[...], axis=-1, keepdims=True)
            asymmetry = jnp.abs(early - late)
            features = jnp.concatenate(
                [max_strength, variance, peak_count, asymmetry], axis=-1)
            h = jnp.dot(features, w1_ref[...],
                        preferred_element_type=jnp.float32) + b1_ref[...]
            mu = jnp.mean(h, axis=-1, keepdims=True)
            var = jnp.mean((h - mu) ** 2, axis=-1, keepdims=True)
            h = (h - mu) / jnp.sqrt(var + LN_EPS) * gamma_ref[...] + beta_ref[...]
            h = 0.5 * h * (1.0 + jax.lax.erf(h * 0.7071067811865476))
            o_ref[...] = jnp.dot(h, w2_ref[...],
                                 preferred_element_type=jnp.float32) + b2_ref[...]


def kernel(x, padding_mask, W1, b1, gamma, beta, W2, b2):
    B, T, D = x.shape
    t_blocks = T // TB
    n_early = (T // 2) // TB
    maskf = padding_mask.astype(jnp.float32)

    body = functools.partial(_detector_kernel, t_blocks=t_blocks,
                             n_early=n_early)
    out = pl.pallas_call(
        body,
        out_shape=jax.ShapeDtypeStruct((B, DF), jnp.float32),
        grid=(B // BB, 2, t_blocks),
        in_specs=[
            pl.BlockSpec((BB, TB, D), lambda bi, ph, ti: (bi, ti, 0)),
            pl.BlockSpec((BB, TB), lambda bi, ph, ti: (bi, ti)),
            pl.BlockSpec((4, DF), lambda bi, ph, ti: (0, 0)),
            pl.BlockSpec((1, DF), lambda bi, ph, ti: (0, 0)),
            pl.BlockSpec((1, DF), lambda bi, ph, ti: (0, 0)),
            pl.BlockSpec((1, DF), lambda bi, ph, ti: (0, 0)),
            pl.BlockSpec((DF, DF), lambda bi, ph, ti: (0, 0)),
            pl.BlockSpec((1, DF), lambda bi, ph, ti: (0, 0)),
        ],
        out_specs=pl.BlockSpec((BB, DF), lambda bi, ph, ti: (bi, 0)),
        scratch_shapes=[
            pltpu.VMEM((BB, D), jnp.float32),
            pltpu.VMEM((BB, D), jnp.float32),
            pltpu.VMEM((BB, D), jnp.float32),
            pltpu.VMEM((BB, 1), jnp.float32),
            pltpu.VMEM((BB, 1), jnp.float32),
            pltpu.VMEM((BB, D), jnp.float32),
        ],
        compiler_params=pltpu.CompilerParams(
            dimension_semantics=("parallel", "arbitrary", "arbitrary"),
            vmem_limit_bytes=56 * 1024 * 1024,
        ),
        name="caustic_detector",
    )(x, maskf, W1, b1.reshape(1, DF), gamma.reshape(1, DF),
      beta.reshape(1, DF), W2, b2.reshape(1, DF))
    return out
